# baseline (speedup 1.0000x reference)
import functools

import jax
import jax.numpy as jnp
from jax import lax
from jax.experimental import pallas as pl
from jax.experimental.pallas import tpu as pltpu
from jax.experimental.pallas import tpu_sc as plsc

D = 32
NW = 32
BB = 128


def _emb_body(table_hbm, idxt_hbm, out_hbm, idx_v, g_v, t_v, gsem, ssem,
              *, nl, bpw, nch):
    nbt = bpw // BB
    wid = lax.axis_index("s") * 2 + lax.axis_index("c")
    b0 = wid * bpw
    pltpu.sync_copy(idxt_hbm.at[:, pl.ds(b0, bpw)], idx_v)

    iota = lax.iota(jnp.int32, 16)

    def start_gather(k, buf):
        l = k // nbt
        bt = k - l * nbt
        pltpu.async_copy(
            table_hbm.at[idx_v.at[l, pl.ds(bt * BB, BB)]], g_v.at[buf],
            gsem.at[buf])

    def wait_gather(buf):
        pltpu.make_async_copy(
            table_hbm.at[idx_v.at[0, pl.ds(0, BB)]], g_v.at[buf],
            gsem.at[buf]).wait()

    def transpose(buf):
        g = g_v.at[buf]
        t = t_v.at[buf]
        for m in range(BB // 16):
            ridx = iota + (m * 16)
            for c in range(D):
                cidx = jnp.full((16,), c, dtype=jnp.int32)
                t[c // 8, c % 8, pl.ds(m * 16, 16)] = plsc.load_gather(
                    g, [ridx, cidx])

    def start_store(k, buf):
        l = k // nbt
        bt = k - l * nbt
        pltpu.async_copy(
            t_v.at[buf], out_hbm.at[l, :, wid * nbt + bt], ssem.at[buf])

    def wait_store(buf):
        pltpu.make_async_copy(
            t_v.at[buf], out_hbm.at[0, :, 0], ssem.at[buf]).wait()

    start_gather(0, 0)
    start_gather(1, 1)

    @pl.loop(0, nch, step=2)
    def _steady(i):
        for b in range(2):
            k = i + b
            wait_gather(b)

            @pl.when(k >= 2)
            def _():
                wait_store(b)

            transpose(b)
            start_store(k, b)

            @pl.when(k + 2 < nch)
            def _():
                start_gather(k + 2, b)

    wait_store(0)
    wait_store(1)


def kernel(x, weight):
    b, nl = x.shape
    n = b * nl
    bpw = b // NW
    nch = (bpw // BB) * nl
    idxt = x.T.astype(jnp.int32)

    mesh = plsc.VectorSubcoreMesh(core_axis_name="c", subcore_axis_name="s")
    body = functools.partial(_emb_body, nl=nl, bpw=bpw, nch=nch)
    out5 = pl.kernel(
        body,
        out_type=jax.ShapeDtypeStruct((nl, D // 8, b // BB, 8, BB),
                                      jnp.float32),
        mesh=mesh,
        scratch_types=[
            pltpu.VMEM((nl, bpw), jnp.int32),
            pltpu.VMEM((2, BB, D), jnp.float32),
            pltpu.VMEM((2, D // 8, 8, BB), jnp.float32),
            pltpu.SemaphoreType.DMA((2,)),
            pltpu.SemaphoreType.DMA((2,)),
        ],
        compiler_params=pltpu.CompilerParams(use_tc_tiling_on_sc=False,
                                             needs_layout_passes=False),
    )(weight, idxt)
    return out5.transpose(2, 4, 0, 1, 3).reshape(b, nl, D)


# speedup vs baseline: 1.2311x; 1.2311x over previous
import functools

import jax
import jax.numpy as jnp
from jax import lax
from jax.experimental import pallas as pl
from jax.experimental.pallas import tpu as pltpu
from jax.experimental.pallas import tpu_sc as plsc

D = 32
NW = 32
BB = 128


def _emb_body(table_hbm, idxt_hbm, out_hbm, idx_v, g_v, t_v, gsem, ssem,
              *, nl, bpw, nch):
    nbt = bpw // BB
    wid = lax.axis_index("s") * 2 + lax.axis_index("c")
    b0 = wid * bpw
    pltpu.sync_copy(idxt_hbm.at[:, pl.ds(b0, bpw)], idx_v)

    iota = lax.iota(jnp.int32, 16)
    lo128 = iota * BB
    hi128 = (iota + 16) * BB

    def start_gather(k, buf):
        l = k // nbt
        bt = k - l * nbt
        pltpu.async_copy(
            table_hbm.at[idx_v.at[l, pl.ds(bt * BB, BB)]], g_v.at[buf],
            gsem.at[buf])

    def wait_gather(buf):
        pltpu.make_async_copy(
            table_hbm.at[idx_v.at[0, pl.ds(0, BB)]], g_v.at[buf],
            gsem.at[buf]).wait()

    def transpose(buf):
        t = t_v.at[buf]
        for b in range(BB):
            va = g_v[buf, b, pl.ds(0, 16)]
            vb = g_v[buf, b, pl.ds(16, 16)]
            plsc.store_scatter(t, [lo128 + b], va)
            plsc.store_scatter(t, [hi128 + b], vb)

    def start_store(k, buf):
        l = k // nbt
        bt = k - l * nbt
        for ct in range(D // 8):
            pltpu.async_copy(
                t_v.at[buf, pl.ds(ct * 8 * BB, 8 * BB)],
                out_hbm.at[l * (D // 8) + ct, wid * nbt + bt],
                ssem.at[buf])

    def wait_store(buf):
        for ct in range(D // 8):
            pltpu.make_async_copy(
                t_v.at[buf, pl.ds(ct * 8 * BB, 8 * BB)],
                out_hbm.at[0, 0], ssem.at[buf]).wait()

    start_gather(0, 0)
    start_gather(1, 1)

    @pl.loop(0, nch, step=2)
    def _steady(i):
        for b in range(2):
            k = i + b
            wait_gather(b)

            @pl.when(k >= 2)
            def _():
                wait_store(b)

            transpose(b)
            start_store(k, b)

            @pl.when(k + 2 < nch)
            def _():
                start_gather(k + 2, b)

    wait_store(0)
    wait_store(1)


def kernel(x, weight):
    b, nl = x.shape
    bpw = b // NW
    nch = (bpw // BB) * nl
    idxt = x.T.astype(jnp.int32)

    mesh = plsc.VectorSubcoreMesh(core_axis_name="c", subcore_axis_name="s")
    body = functools.partial(_emb_body, nl=nl, bpw=bpw, nch=nch)
    out3 = pl.kernel(
        body,
        out_type=jax.ShapeDtypeStruct((nl * (D // 8), b // BB, 8 * BB),
                                      jnp.float32),
        mesh=mesh,
        scratch_types=[
            pltpu.VMEM((nl, bpw), jnp.int32),
            pltpu.VMEM((2, BB, D), jnp.float32),
            pltpu.VMEM((2, D * BB), jnp.float32),
            pltpu.SemaphoreType.DMA((2,)),
            pltpu.SemaphoreType.DMA((2,)),
        ],
        compiler_params=pltpu.CompilerParams(use_tc_tiling_on_sc=False,
                                             needs_layout_passes=False),
    )(weight, idxt)
    out5 = out3.reshape(nl, D // 8, b // BB, 8, BB)
    return out5.transpose(2, 4, 0, 1, 3).reshape(b, nl, D)


# speedup vs baseline: 1.5928x; 1.2937x over previous
import functools

import jax
import jax.numpy as jnp
from jax import lax
from jax.experimental import pallas as pl
from jax.experimental.pallas import tpu as pltpu
from jax.experimental.pallas import tpu_sc as plsc

D = 32
NW = 32
BB = 128


def _emb_body(table_hbm, idxt_hbm, out_hbm, idx_v, g_v, t_v, gsem, ssem,
              *, nl, bpw, nch):
    nbt = bpw // BB
    wid = lax.axis_index("s") * 2 + lax.axis_index("c")
    b0 = wid * bpw
    pltpu.sync_copy(idxt_hbm.at[:, pl.ds(b0, bpw)], idx_v)

    iota = lax.iota(jnp.int32, 16)

    def start_gather(k, buf):
        l = k // nbt
        bt = k - l * nbt
        pltpu.async_copy(
            table_hbm.at[idx_v.at[l, pl.ds(bt * BB, BB)]], g_v.at[buf],
            gsem.at[buf])

    def wait_gather(buf):
        pltpu.make_async_copy(
            table_hbm.at[idx_v.at[0, pl.ds(0, BB)]], g_v.at[buf],
            gsem.at[buf]).wait()

    def transpose(buf):
        t = t_v.at[buf]
        for b in range(BB):
            va = g_v[buf, b, pl.ds(0, 16)]
            vb = g_v[buf, b, pl.ds(16, 16)]
            bvec = jnp.full((16,), b, dtype=jnp.int32)
            plsc.store_scatter(t, [iota, bvec], va)
            plsc.store_scatter(t, [iota + 16, bvec], vb)

    def start_store(k, buf):
        l = k // nbt
        bt = k - l * nbt
        for ct in range(D // 8):
            pltpu.async_copy(
                t_v.at[buf, pl.ds(ct * 8, 8), pl.ds(0, BB)],
                out_hbm.at[l * (D // 8) + ct, wid * nbt + bt],
                ssem.at[buf])

    def wait_store(buf):
        for ct in range(D // 8):
            pltpu.make_async_copy(
                t_v.at[buf, pl.ds(ct * 8, 8), pl.ds(0, BB)],
                out_hbm.at[0, 0], ssem.at[buf]).wait()

    start_gather(0, 0)
    start_gather(1, 1)

    @pl.loop(0, nch, step=2)
    def _steady(i):
        for b in range(2):
            k = i + b
            wait_gather(b)

            @pl.when(k >= 2)
            def _():
                wait_store(b)

            transpose(b)
            start_store(k, b)

            @pl.when(k + 2 < nch)
            def _():
                start_gather(k + 2, b)

    wait_store(0)
    wait_store(1)


def kernel(x, weight):
    b, nl = x.shape
    bpw = b // NW
    nch = (bpw // BB) * nl
    idxt = x.T.astype(jnp.int32)

    mesh = plsc.VectorSubcoreMesh(core_axis_name="c", subcore_axis_name="s")
    body = functools.partial(_emb_body, nl=nl, bpw=bpw, nch=nch)
    out4 = pl.kernel(
        body,
        out_type=jax.ShapeDtypeStruct((nl * (D // 8), b // BB, 8, BB),
                                      jnp.float32),
        mesh=mesh,
        scratch_types=[
            pltpu.VMEM((nl, bpw), jnp.int32),
            pltpu.VMEM((2, BB, D), jnp.float32),
            pltpu.VMEM((2, D, 137), jnp.float32),
            pltpu.SemaphoreType.DMA((2,)),
            pltpu.SemaphoreType.DMA((2,)),
        ],
        compiler_params=pltpu.CompilerParams(use_tc_tiling_on_sc=False,
                                             needs_layout_passes=False),
    )(weight, idxt)
    out5 = out4.reshape(nl, D // 8, b // BB, 8, BB)
    return out5.transpose(2, 4, 0, 1, 3).reshape(b, nl, D)
